# baseline (speedup 1.0000x reference)
import functools
import math

import jax
import jax.numpy as jnp
from jax import lax
from jax.experimental import pallas as pl
from jax.experimental.pallas import tpu as pltpu
from jax.experimental.pallas import tpu_sc as plsc

N = 10000
E = 160000
H = 128
S = 16
EF = 16
K = 2
W = 2 * H
GW = 3 * H

NC = 2
NS = 16
NW = NC * NS
L = 16
EPW = E // NW
CH = 32
NCHUNK = (EPW // CH)
EPI = EPW - NCHUNK * CH
DUMMY = N
NPAD = N + L
EPAD = 5120
EFULL = NW * EPAD
SPADN = 10112
INV_TEMP = 1.0 / math.sqrt(float(H))

_NBLK = 1000
_EBLK = 2000

@functools.cache
def _sc_mesh():
    return plsc.VectorSubcoreMesh(
        core_axis_name="c", subcore_axis_name="s",
        num_cores=NC, num_subcores=NS)


def _ln(h, lnw, lnb):
    mu = jnp.mean(h, axis=-1, keepdims=True)
    var = jnp.mean((h - mu) * (h - mu), axis=-1, keepdims=True)
    return (h - mu) * lax.rsqrt(var + 1e-5) * lnw + lnb


def _proj_first_body(x_ref, xs_ref, lnw_ref, lnb_ref, wh1_ref, ws1_ref,
                     wh2_ref, ws2_ref, hn_ref, p1_ref, p2_ref):
    hn = _ln(x_ref[...], lnw_ref[...], lnb_ref[...])
    hn_ref[...] = hn
    xs = xs_ref[...]
    p1_ref[...] = (jnp.dot(hn, wh1_ref[...], preferred_element_type=jnp.float32)
                   + jnp.dot(xs, ws1_ref[...], preferred_element_type=jnp.float32))
    p2_ref[...] = (jnp.dot(hn, wh2_ref[...], preferred_element_type=jnp.float32)
                   + jnp.dot(xs, ws2_ref[...], preferred_element_type=jnp.float32))


def _proj_next_body(x_ref, hp_ref, xs_ref, lnw_ref, lnb_ref, wh1_ref, ws1_ref,
                    wh2_ref, ws2_ref, hn_ref, p1_ref, p2_ref, h_ref):
    h = x_ref[...] + hp_ref[0] + hp_ref[1]
    h_ref[...] = h
    hn = _ln(h, lnw_ref[...], lnb_ref[...])
    hn_ref[...] = hn
    xs = xs_ref[...]
    p1_ref[...] = (jnp.dot(hn, wh1_ref[...], preferred_element_type=jnp.float32)
                   + jnp.dot(xs, ws1_ref[...], preferred_element_type=jnp.float32))
    p2_ref[...] = (jnp.dot(hn, wh2_ref[...], preferred_element_type=jnp.float32)
                   + jnp.dot(xs, ws2_ref[...], preferred_element_type=jnp.float32))


def _proj_specs():
    full = lambda shape: pl.BlockSpec(shape, lambda i: (0,) * len(shape))
    nb = lambda w: pl.BlockSpec((_NBLK, w), lambda i: (i, 0))
    return full, nb


def _proj_first(x, xs, lnw, lnb, wh1, ws1, wh2, ws2):
    full, nb = _proj_specs()
    return pl.pallas_call(
        _proj_first_body,
        grid=(N // _NBLK,),
        in_specs=[nb(H), nb(S), full((1, H)), full((1, H)),
                  full((H, W)), full((S, W)), full((H, W)), full((S, W))],
        out_specs=[nb(H), nb(W), nb(W)],
        out_shape=[jax.ShapeDtypeStruct((N, H), jnp.float32),
                   jax.ShapeDtypeStruct((N, W), jnp.float32),
                   jax.ShapeDtypeStruct((N, W), jnp.float32)],
    )(x, xs, lnw.reshape(1, H), lnb.reshape(1, H), wh1, ws1, wh2, ws2)


def _proj_next(x, hp, xs, lnw, lnb, wh1, ws1, wh2, ws2):
    full, nb = _proj_specs()
    return pl.pallas_call(
        _proj_next_body,
        grid=(N // _NBLK,),
        in_specs=[nb(H), pl.BlockSpec((2, _NBLK, H), lambda i: (0, i, 0)),
                  nb(S), full((1, H)), full((1, H)),
                  full((H, W)), full((S, W)), full((H, W)), full((S, W))],
        out_specs=[nb(H), nb(W), nb(W), nb(H)],
        out_shape=[jax.ShapeDtypeStruct((N, H), jnp.float32),
                   jax.ShapeDtypeStruct((N, W), jnp.float32),
                   jax.ShapeDtypeStruct((N, W), jnp.float32),
                   jax.ShapeDtypeStruct((N, H), jnp.float32)],
    )(x, hp, xs, lnw.reshape(1, H), lnb.reshape(1, H), wh1, ws1, wh2, ws2)


def _cpre_body(ef_ref, w0_ref, b0_ref, w1_ref, b1_ref, w2_ref, b2_ref,
               w3_ref, b3_ref, c0_ref, c1_ref, c2_ref, c3_ref):
    ef = ef_ref[...]
    c0_ref[...] = jnp.dot(ef, w0_ref[...], preferred_element_type=jnp.float32) + b0_ref[...]
    c1_ref[...] = jnp.dot(ef, w1_ref[...], preferred_element_type=jnp.float32) + b1_ref[...]
    c2_ref[...] = jnp.dot(ef, w2_ref[...], preferred_element_type=jnp.float32) + b2_ref[...]
    c3_ref[...] = jnp.dot(ef, w3_ref[...], preferred_element_type=jnp.float32) + b3_ref[...]


def _cpre(ef, ws, bs):
    full = lambda shape: pl.BlockSpec(shape, lambda i: (0, 0))
    eb = lambda w: pl.BlockSpec((_EBLK, w), lambda i: (i, 0))
    args = [ef]
    for wmat, bvec in zip(ws, bs):
        args.append(wmat)
        args.append(bvec.reshape(1, W))
    return pl.pallas_call(
        _cpre_body,
        grid=(E // _EBLK,),
        in_specs=[eb(EF)] + [full((EF, W)), full((1, W))] * 4,
        out_specs=[eb(W)] * 4,
        out_shape=[jax.ShapeDtypeStruct((E, W), jnp.float32)] * 4,
    )(*args)


def _gru_body(x_ref, h1f_ref, hpf_ref, h1r_ref, hpr_ref,
              rw1_ref, rb1_ref, rw2_ref, rb2_ref,
              zw1_ref, zb1_ref, zw2_ref, zb2_ref,
              cw1_ref, cb1_ref, cw2_ref, cb2_ref,
              fin_ref, z_ref, r_ref):
    x = x_ref[...]
    mf = h1f_ref[...] + hpf_ref[0] + hpf_ref[1] - x
    mr = h1r_ref[...] + hpr_ref[0] + hpr_ref[1] - x
    gi = jnp.concatenate([x, mf, mr], axis=-1)

    def mlp(inp, w1, b1, w2, b2):
        hh = jnp.maximum(
            jnp.dot(inp, w1[...], preferred_element_type=jnp.float32) + b1[...],
            0.0)
        return jnp.dot(hh, w2[...], preferred_element_type=jnp.float32) + b2[...]

    r = jax.nn.sigmoid(mlp(gi, rw1_ref, rb1_ref, rw2_ref, rb2_ref))
    z = jax.nn.sigmoid(mlp(gi, zw1_ref, zb1_ref, zw2_ref, zb2_ref))
    ci = jnp.concatenate([r * x, mf, mr], axis=-1)
    cand = jnp.tanh(mlp(ci, cw1_ref, cb1_ref, cw2_ref, cb2_ref))
    fin_ref[...] = (1.0 - z) * x + z * cand
    z_ref[...] = z
    r_ref[...] = r


def _gru(x, h1f, hpf, h1r, hpr, rw1, rb1, rw2, rb2, zw1, zb1, zw2, zb2,
         cw1, cb1, cw2, cb2):
    full = lambda shape: pl.BlockSpec(shape, lambda i: (0,) * len(shape))
    nb = lambda w: pl.BlockSpec((_NBLK, w), lambda i: (i, 0))
    pb = pl.BlockSpec((2, _NBLK, H), lambda i: (0, i, 0))
    return pl.pallas_call(
        _gru_body,
        grid=(N // _NBLK,),
        in_specs=[nb(H), nb(H), pb, nb(H), pb,
                  full((GW, GW)), full((1, GW)), full((GW, H)), full((1, H)),
                  full((GW, GW)), full((1, GW)), full((GW, H)), full((1, H)),
                  full((GW, GW)), full((1, GW)), full((GW, H)), full((1, H))],
        out_specs=[nb(H), nb(H), nb(H)],
        out_shape=[jax.ShapeDtypeStruct((N, H), jnp.float32)] * 3,
    )(x, h1f, hpf, h1r, hpr,
      rw1.T, rb1.reshape(1, GW), rw2.T, rb2.reshape(1, H),
      zw1.T, zb1.reshape(1, GW), zw2.T, zb2.reshape(1, H),
      cw1.T, cb1.reshape(1, GW), cw2.T, cb2.reshape(1, H))


_LANES = None


CH2 = 64


def _gat_body(p1_h, p2_h, a_h, b_h, g1_h, g2_h,
              a_v, b_v, st1, st2, st1b, st2b, sem):
    cid = lax.axis_index("c")
    sid = lax.axis_index("s")
    wid = sid * NC + cid
    base = pl.multiple_of(wid * EPW, 8)
    pbase = pl.multiple_of(wid * EPAD, 128)

    pltpu.sync_copy(a_h.at[pl.ds(pbase, EPAD)], a_v)
    pltpu.sync_copy(b_h.at[pl.ds(pbase, EPAD)], b_v)

    def _pair(tp, carry):
        e0 = pl.multiple_of(tp * 2 * CH2, 8)
        e1 = pl.multiple_of((tp * 2 + 1) * CH2, 8)
        ga1 = pltpu.async_copy(p1_h.at[a_v.at[pl.ds(e0, CH2)]], st1, sem)
        ga2 = pltpu.async_copy(p2_h.at[b_v.at[pl.ds(e0, CH2)]], st2, sem)
        gb1 = pltpu.async_copy(p1_h.at[a_v.at[pl.ds(e1, CH2)]], st1b, sem)
        gb2 = pltpu.async_copy(p2_h.at[b_v.at[pl.ds(e1, CH2)]], st2b, sem)
        ga1.wait(); ga2.wait()
        pltpu.sync_copy(st1, g1_h.at[pl.ds(base + e0, CH2)])
        pltpu.sync_copy(st2, g2_h.at[pl.ds(base + e0, CH2)])
        gb1.wait(); gb2.wait()
        pltpu.sync_copy(st1b, g1_h.at[pl.ds(base + e1, CH2)])
        pltpu.sync_copy(st2b, g2_h.at[pl.ds(base + e1, CH2)])
        return carry
    lax.fori_loop(0, EPW // CH2 // 2, _pair, 0)

    ebase = (EPW // CH2) * CH2
    pltpu.async_copy(p1_h.at[a_v.at[pl.ds(ebase, L)]],
                     st1.at[pl.ds(0, L)], sem).wait()
    pltpu.async_copy(p2_h.at[b_v.at[pl.ds(ebase, L)]],
                     st2.at[pl.ds(0, L)], sem).wait()
    pltpu.sync_copy(st1.at[pl.ds(0, EPI)], g1_h.at[pl.ds(base + ebase, EPI)])
    pltpu.sync_copy(st2.at[pl.ds(0, EPI)], g2_h.at[pl.ds(base + ebase, EPI)])


def _gat(p1, p2, a_p, b_p):
    return pl.kernel(
        _gat_body,
        out_type=[jax.ShapeDtypeStruct((E, W), jnp.float32),
                  jax.ShapeDtypeStruct((E, W), jnp.float32)],
        mesh=_sc_mesh(),
        compiler_params=pltpu.CompilerParams(needs_layout_passes=False),
        scratch_types=[
            pltpu.VMEM((EPAD,), jnp.int32),
            pltpu.VMEM((EPAD,), jnp.int32),
            pltpu.VMEM((CH2, W), jnp.float32),
            pltpu.VMEM((CH2, W), jnp.float32),
            pltpu.VMEM((CH2, W), jnp.float32),
            pltpu.VMEM((CH2, W), jnp.float32),
            pltpu.SemaphoreType.DMA,
        ],
    )(p1, p2, a_p, b_p)


def _tscore_body(g1_ref, g2_ref, c_ref, w2_ref, b2_ref, out_ref, *, is_fwd):
    pre = g1_ref[...] + g2_ref[...] + c_ref[...]
    t = jnp.sum(jnp.maximum(pre, 0.0) * w2_ref[...], axis=-1, keepdims=True) \
        + b2_ref[0, 0]
    if is_fwd:
        t = jnp.where(t >= 0.0, t, 0.01 * t)
        out_ref[...] = jnp.exp(t * INV_TEMP)
    else:
        out_ref[...] = 1.0 / (1.0 + jnp.exp(-t))


def _tscore(g1, g2, c, w2, b2, is_fwd):
    full = lambda shape: pl.BlockSpec(shape, lambda i: (0, 0))
    eb = lambda w: pl.BlockSpec((_EBLK, w), lambda i: (i, 0))
    out = pl.pallas_call(
        functools.partial(_tscore_body, is_fwd=is_fwd),
        grid=(E // _EBLK,),
        in_specs=[eb(W), eb(W), eb(W), full((1, W)), full((1, 1))],
        out_specs=pl.BlockSpec((_EBLK, 1), lambda i: (i, 0)),
        out_shape=jax.ShapeDtypeStruct((E, 1), jnp.float32),
    )(g1, g2, c, w2.reshape(1, W), b2.reshape(1, 1))
    return out[:, 0]


def _sden_body(e_h, dst_h, sp_h, e_v, dst_v, s_loc):
    cid = lax.axis_index("c")
    sid = lax.axis_index("s")
    wid = sid * NC + cid
    pbase = pl.multiple_of(wid * EPAD, 128)
    lanes = lax.iota(jnp.int32, L)

    pltpu.sync_copy(e_h.at[pl.ds(pbase, EPAD)], e_v)
    pltpu.sync_copy(dst_h.at[pl.ds(pbase, EPAD)], dst_v)

    zero16 = jnp.zeros((L,), jnp.float32)

    def _zero(k, carry):
        s_loc[pl.ds(pl.multiple_of(k * L, 8), L)] = zero16
        return carry
    lax.fori_loop(0, SPADN // L, _zero, 0)

    def _grp(g, carry):
        off = pl.multiple_of(g * L, 8)
        ev = e_v[pl.ds(off, L)]
        d16 = dst_v[pl.ds(off, L)]
        for k in range(L):
            dk = jnp.full((L,), d16[k], jnp.int32)
            plsc.addupdate_scatter(s_loc, [dk], ev, mask=(lanes == k))
        return carry
    lax.fori_loop(0, EPW // L, _grp, 0)

    ebase = (EPW // L) * L
    ev = e_v[pl.ds(ebase, L)]
    d16 = jnp.where(lanes < EPI, dst_v[pl.ds(ebase, L)], DUMMY)
    for k in range(EPI):
        dk = jnp.full((L,), d16[k], jnp.int32)
        plsc.addupdate_scatter(s_loc, [dk], ev, mask=(lanes == k))

    pltpu.sync_copy(s_loc, sp_h.at[pl.ds(pl.multiple_of(wid * SPADN, 128),
                                         SPADN)])


def _sden(e_p, dst_p):
    return pl.kernel(
        _sden_body,
        out_type=jax.ShapeDtypeStruct((NW * SPADN,), jnp.float32),
        mesh=_sc_mesh(),
        compiler_params=pltpu.CompilerParams(needs_layout_passes=False),
        scratch_types=[
            pltpu.VMEM((EPAD,), jnp.float32),
            pltpu.VMEM((EPAD,), jnp.int32),
            pltpu.VMEM((SPADN,), jnp.float32),
        ],
    )(e_p, dst_p)


_ZROWS = 16


def _scale_scatter(hrows, w16, row0, idx16, h_acc, sem):
    for k in range(L):
        wk = w16[k]
        for rr in range(H // L):
            sl = (row0 + k, pl.ds(rr * L, L))
            hrows[sl] = hrows[sl] * wk
    return pltpu.async_copy(hrows.at[pl.ds(row0, L)], h_acc.at[idx16], sem,
                            add=True)


def _zero_acc(zbuf, h_acc, sid):
    zero16 = jnp.zeros((L,), jnp.float32)

    def _zb(i, carry):
        for jj in range(H // L):
            zbuf[i, pl.ds(jj * L, L)] = zero16
        return carry
    lax.fori_loop(0, _ZROWS, _zb, 0)
    stripe = sid * (NPAD // NS)
    for q in range(NPAD // NS // _ZROWS):
        pltpu.sync_copy(zbuf, h_acc.at[pl.ds(stripe + q * _ZROWS, _ZROWS)])
    pltpu.sync_copy(zbuf, h_acc.at[pl.ds(stripe + NPAD // NS - _ZROWS,
                                         _ZROWS)])


def _f2_body(e_h, sp_h, hn_h, src_h, dst_h,
             w_h, hp_h,
             e_v, src_v, dst_v, s_tot, s_tmp, hrows, w_v, zbuf, h_acc,
             sem, sem2):
    cid = lax.axis_index("c")
    sid = lax.axis_index("s")
    wid = sid * NC + cid
    base = pl.multiple_of(wid * EPW, 8)
    pbase = pl.multiple_of(wid * EPAD, 128)
    lanes = lax.iota(jnp.int32, L)

    pltpu.sync_copy(e_h.at[pl.ds(pbase, EPAD)], e_v)
    pltpu.sync_copy(src_h.at[pl.ds(pbase, EPAD)], src_v)
    pltpu.sync_copy(dst_h.at[pl.ds(pbase, EPAD)], dst_v)

    _zero_acc(zbuf, h_acc, sid)
    plsc.subcore_barrier()

    pltpu.sync_copy(sp_h.at[pl.ds(0, SPADN)], s_tot)
    for rpart in range(1, NW):
        pltpu.sync_copy(sp_h.at[pl.ds(rpart * SPADN, SPADN)], s_tmp)

        def _acc(k2, carry):
            off = pl.ds(pl.multiple_of(k2 * L, 8), L)
            s_tot[off] = s_tot[off] + s_tmp[off]
            return carry
        lax.fori_loop(0, SPADN // L, _acc, 0)

    def _chunk(t, carry):
        eb = pl.multiple_of(t * CH, 8)
        pltpu.async_copy(hn_h.at[src_v.at[pl.ds(eb, CH)]], hrows, sem).wait()
        scat = []
        for g in range(CH // L):
            d16 = dst_v[pl.ds(eb + g * L, L)]
            e16 = e_v[pl.ds(eb + g * L, L)]
            s16 = plsc.load_gather(s_tot, [d16])
            w16 = e16 / (s16 + 1e-9)
            w_v[pl.ds(eb + g * L, L)] = w16
            scat.append(_scale_scatter(hrows, w16, g * L, d16, h_acc, sem2))
        for sc in scat:
            sc.wait()
        return carry
    lax.fori_loop(0, NCHUNK, _chunk, 0)

    ebase = NCHUNK * CH
    msk = lanes < EPI
    vsrc = jnp.where(msk, src_v[pl.ds(ebase, L)], 0)
    vdst_raw = dst_v[pl.ds(ebase, L)]
    vdst = jnp.where(msk, vdst_raw, 0)
    pltpu.async_copy(hn_h.at[vsrc], hrows.at[pl.ds(0, L)], sem).wait()
    e16 = e_v[pl.ds(ebase, L)]
    s16 = plsc.load_gather(s_tot, [vdst])
    w16 = e16 / (s16 + 1e-9)
    w_v[pl.ds(ebase, L)] = w16
    dsc = jnp.where(msk, vdst_raw, DUMMY)
    _scale_scatter(hrows, w16, 0, dsc, h_acc, sem2).wait()

    pltpu.sync_copy(w_v, w_h.at[pl.ds(pbase, EPAD)])
    plsc.subcore_barrier()

    @pl.when(sid == 0)
    def _flush():
        pltpu.sync_copy(h_acc.at[pl.ds(0, N)], hp_h.at[cid])


def _f2(e, sp, hn, srci, dsti):
    return pl.kernel(
        _f2_body,
        out_type=[jax.ShapeDtypeStruct((EFULL,), jnp.float32),
                  jax.ShapeDtypeStruct((2, N, H), jnp.float32)],
        mesh=_sc_mesh(),
        compiler_params=pltpu.CompilerParams(needs_layout_passes=False),
        scratch_types=[
            pltpu.VMEM((EPAD,), jnp.float32),
            pltpu.VMEM((EPAD,), jnp.int32),
            pltpu.VMEM((EPAD,), jnp.int32),
            pltpu.VMEM((SPADN,), jnp.float32),
            pltpu.VMEM((SPADN,), jnp.float32),
            pltpu.VMEM((CH, H), jnp.float32),
            pltpu.VMEM((EPAD,), jnp.float32),
            pltpu.VMEM((_ZROWS, H), jnp.float32),
            pltpu.VMEM_SHARED((NPAD, H), jnp.float32),
            pltpu.SemaphoreType.DMA,
            pltpu.SemaphoreType.DMA,
        ],
    )(e, sp, hn, srci, dsti)


def _rsc_body(g_h, hn_h, gat_h, sct_h, hp_h,
              g_v, gat_v, sct_v, hrows, zbuf, h_acc, sem, sem2):
    cid = lax.axis_index("c")
    sid = lax.axis_index("s")
    wid = sid * NC + cid
    pbase = pl.multiple_of(wid * EPAD, 128)
    lanes = lax.iota(jnp.int32, L)

    pltpu.sync_copy(g_h.at[pl.ds(pbase, EPAD)], g_v)
    pltpu.sync_copy(gat_h.at[pl.ds(pbase, EPAD)], gat_v)
    pltpu.sync_copy(sct_h.at[pl.ds(pbase, EPAD)], sct_v)

    _zero_acc(zbuf, h_acc, sid)
    plsc.subcore_barrier()

    def _chunk(t, carry):
        eb = pl.multiple_of(t * CH, 8)
        pltpu.async_copy(hn_h.at[gat_v.at[pl.ds(eb, CH)]], hrows, sem).wait()
        scat = []
        for g in range(CH // L):
            w16 = g_v[pl.ds(eb + g * L, L)]
            i16 = sct_v[pl.ds(eb + g * L, L)]
            scat.append(_scale_scatter(hrows, w16, g * L, i16, h_acc, sem2))
        for sc in scat:
            sc.wait()
        return carry
    lax.fori_loop(0, NCHUNK, _chunk, 0)

    ebase = NCHUNK * CH
    msk = lanes < EPI
    vgat = jnp.where(msk, gat_v[pl.ds(ebase, L)], 0)
    pltpu.async_copy(hn_h.at[vgat], hrows.at[pl.ds(0, L)], sem).wait()
    w16 = g_v[pl.ds(ebase, L)]
    ssc = jnp.where(msk, sct_v[pl.ds(ebase, L)], DUMMY)
    _scale_scatter(hrows, w16, 0, ssc, h_acc, sem2).wait()

    plsc.subcore_barrier()

    @pl.when(sid == 0)
    def _flush():
        pltpu.sync_copy(h_acc.at[pl.ds(0, N)], hp_h.at[cid])


def _rsc(g_p, hn, gat_p, sct_p):
    return pl.kernel(
        _rsc_body,
        out_type=jax.ShapeDtypeStruct((2, N, H), jnp.float32),
        mesh=_sc_mesh(),
        compiler_params=pltpu.CompilerParams(needs_layout_passes=False),
        scratch_types=[
            pltpu.VMEM((EPAD,), jnp.float32),
            pltpu.VMEM((EPAD,), jnp.int32),
            pltpu.VMEM((EPAD,), jnp.int32),
            pltpu.VMEM((CH, H), jnp.float32),
            pltpu.VMEM((_ZROWS, H), jnp.float32),
            pltpu.VMEM_SHARED((NPAD, H), jnp.float32),
            pltpu.SemaphoreType.DMA,
            pltpu.SemaphoreType.DMA,
        ],
    )(g_p, hn, gat_p, sct_p)


def kernel(x, x_s, edge_index, edge_features, fwd_W1, fwd_b1, fwd_W2, fwd_b2,
           rev_W1, rev_b1, rev_W2, rev_b2, ln_w, ln_b, r_W1, r_b1, r_W2, r_b2,
           z_W1, z_b1, z_W2, z_b2, c_W1, c_b1, c_W2, c_b2):
    src = edge_index[0].astype(jnp.int32)
    dst = edge_index[1].astype(jnp.int32)
    pad = lambda a: jnp.pad(a.reshape(NW, EPW),
                            ((0, 0), (0, EPAD - EPW))).reshape(-1)
    unpad = lambda a: a.reshape(NW, EPAD)[:, :EPW].reshape(E)
    src_p, dst_p = pad(src), pad(dst)

    cs = _cpre(edge_features,
               [fwd_W1[0, :, W + 2 * S:].T, fwd_W1[1, :, W + 2 * S:].T,
                rev_W1[0, :, W + 2 * S:].T, rev_W1[1, :, W + 2 * S:].T],
               [fwd_b1[0], fwd_b1[1], rev_b1[0], rev_b1[1]])
    c_f, c_r = cs[:2], cs[2:]

    def wslices(w1):
        return (w1[:, :H].T, w1[:, W:W + S].T,
                w1[:, H:W].T, w1[:, W + S:W + 2 * S].T)

    hn0, p1, p2 = _proj_first(x, x_s, ln_w, ln_b, *wslices(fwd_W1[0]))
    hn0r, q1, q2 = _proj_first(x, x_s, ln_w, ln_b, *wslices(rev_W1[0]))
    g1a, g2a = _gat(p1, p2, src_p, dst_p)
    r1a, r2a = _gat(q1, q2, dst_p, src_p)
    e0 = pad(_tscore(g1a, g2a, c_f[0], fwd_W2[0, 0], fwd_b2[0], True))
    ge0 = pad(_tscore(r1a, r2a, c_r[0], rev_W2[0, 0], rev_b2[0], False))
    sp0 = _sden(e0, dst_p)
    w0, hp0 = _f2(e0, sp0, hn0, src_p, dst_p)
    hq0 = _rsc(ge0, hn0r, dst_p, src_p)

    hn1, p1b, p2b, h1f = _proj_next(x, hp0, x_s, ln_w, ln_b,
                                    *wslices(fwd_W1[1]))
    hn1r, q1c, q2c, h1r = _proj_next(x, hq0, x_s, ln_w, ln_b,
                                     *wslices(rev_W1[1]))
    g1b, g2b = _gat(p1b, p2b, src_p, dst_p)
    r1b, r2b = _gat(q1c, q2c, dst_p, src_p)
    e1 = pad(_tscore(g1b, g2b, c_f[1], fwd_W2[1, 0], fwd_b2[1], True))
    ge1 = pad(_tscore(r1b, r2b, c_r[1], rev_W2[1, 0], rev_b2[1], False))
    sp1 = _sden(e1, dst_p)
    w1, hp1 = _f2(e1, sp1, hn1, src_p, dst_p)
    hq1 = _rsc(ge1, hn1r, dst_p, src_p)
    fwd_ws = jnp.stack([unpad(w0), unpad(w1)], axis=-1)
    rev_ws = jnp.stack([unpad(ge0), unpad(ge1)], axis=-1)

    final, z, r = _gru(x, h1f, hp1, h1r, hq1,
                       r_W1, r_b1, r_W2, r_b2,
                       z_W1, z_b1, z_W2, z_b2,
                       c_W1, c_b1, c_W2, c_b2)
    return (final, fwd_ws, rev_ws, z, r)


# speedup vs baseline: 1.1159x; 1.1159x over previous
import functools
import math

import jax
import jax.numpy as jnp
from jax import lax
from jax.experimental import pallas as pl
from jax.experimental.pallas import tpu as pltpu
from jax.experimental.pallas import tpu_sc as plsc

N = 10000
E = 160000
H = 128
S = 16
EF = 16
K = 2
W = 2 * H
GW = 3 * H

NC = 2
NS = 16
NW = NC * NS
L = 16
EPW = E // NW
CH = 32
NCHUNK = (EPW // CH)
EPI = EPW - NCHUNK * CH
DUMMY = N
NPAD = N + L
EPAD = 5120
EFULL = NW * EPAD
SPADN = 10112
INV_TEMP = 1.0 / math.sqrt(float(H))

_NBLK = 1000
_EBLK = 2000

@functools.cache
def _sc_mesh():
    return plsc.VectorSubcoreMesh(
        core_axis_name="c", subcore_axis_name="s",
        num_cores=NC, num_subcores=NS)


def _ln(h, lnw, lnb):
    mu = jnp.mean(h, axis=-1, keepdims=True)
    var = jnp.mean((h - mu) * (h - mu), axis=-1, keepdims=True)
    return (h - mu) * lax.rsqrt(var + 1e-5) * lnw + lnb


def _proj_first_body(x_ref, xs_ref, lnw_ref, lnb_ref, wh1_ref, ws1_ref,
                     wh2_ref, ws2_ref, hn_ref, p1_ref, p2_ref):
    hn = _ln(x_ref[...], lnw_ref[...], lnb_ref[...])
    hn_ref[...] = hn
    xs = xs_ref[...]
    p1_ref[...] = (jnp.dot(hn, wh1_ref[...], preferred_element_type=jnp.float32)
                   + jnp.dot(xs, ws1_ref[...], preferred_element_type=jnp.float32))
    p2_ref[...] = (jnp.dot(hn, wh2_ref[...], preferred_element_type=jnp.float32)
                   + jnp.dot(xs, ws2_ref[...], preferred_element_type=jnp.float32))


def _proj_next_body(x_ref, hp_ref, xs_ref, lnw_ref, lnb_ref, wh1_ref, ws1_ref,
                    wh2_ref, ws2_ref, hn_ref, p1_ref, p2_ref, h_ref):
    h = x_ref[...] + hp_ref[0] + hp_ref[1]
    h_ref[...] = h
    hn = _ln(h, lnw_ref[...], lnb_ref[...])
    hn_ref[...] = hn
    xs = xs_ref[...]
    p1_ref[...] = (jnp.dot(hn, wh1_ref[...], preferred_element_type=jnp.float32)
                   + jnp.dot(xs, ws1_ref[...], preferred_element_type=jnp.float32))
    p2_ref[...] = (jnp.dot(hn, wh2_ref[...], preferred_element_type=jnp.float32)
                   + jnp.dot(xs, ws2_ref[...], preferred_element_type=jnp.float32))


def _proj_specs():
    full = lambda shape: pl.BlockSpec(shape, lambda i: (0,) * len(shape))
    nb = lambda w: pl.BlockSpec((_NBLK, w), lambda i: (i, 0))
    return full, nb


def _proj_first(x, xs, lnw, lnb, wh1, ws1, wh2, ws2):
    full, nb = _proj_specs()
    return pl.pallas_call(
        _proj_first_body,
        grid=(N // _NBLK,),
        in_specs=[nb(H), nb(S), full((1, H)), full((1, H)),
                  full((H, W)), full((S, W)), full((H, W)), full((S, W))],
        out_specs=[nb(H), nb(W), nb(W)],
        out_shape=[jax.ShapeDtypeStruct((N, H), jnp.float32),
                   jax.ShapeDtypeStruct((N, W), jnp.float32),
                   jax.ShapeDtypeStruct((N, W), jnp.float32)],
    )(x, xs, lnw.reshape(1, H), lnb.reshape(1, H), wh1, ws1, wh2, ws2)


def _proj_next(x, hp, xs, lnw, lnb, wh1, ws1, wh2, ws2):
    full, nb = _proj_specs()
    return pl.pallas_call(
        _proj_next_body,
        grid=(N // _NBLK,),
        in_specs=[nb(H), pl.BlockSpec((2, _NBLK, H), lambda i: (0, i, 0)),
                  nb(S), full((1, H)), full((1, H)),
                  full((H, W)), full((S, W)), full((H, W)), full((S, W))],
        out_specs=[nb(H), nb(W), nb(W), nb(H)],
        out_shape=[jax.ShapeDtypeStruct((N, H), jnp.float32),
                   jax.ShapeDtypeStruct((N, W), jnp.float32),
                   jax.ShapeDtypeStruct((N, W), jnp.float32),
                   jax.ShapeDtypeStruct((N, H), jnp.float32)],
    )(x, hp, xs, lnw.reshape(1, H), lnb.reshape(1, H), wh1, ws1, wh2, ws2)


def _gru_body(x_ref, h1f_ref, hpf_ref, h1r_ref, hpr_ref,
              rw1_ref, rb1_ref, rw2_ref, rb2_ref,
              zw1_ref, zb1_ref, zw2_ref, zb2_ref,
              cw1_ref, cb1_ref, cw2_ref, cb2_ref,
              fin_ref, z_ref, r_ref):
    x = x_ref[...]
    mf = h1f_ref[...] + hpf_ref[0] + hpf_ref[1] - x
    mr = h1r_ref[...] + hpr_ref[0] + hpr_ref[1] - x
    gi = jnp.concatenate([x, mf, mr], axis=-1)

    def mlp(inp, w1, b1, w2, b2):
        hh = jnp.maximum(
            jnp.dot(inp, w1[...], preferred_element_type=jnp.float32) + b1[...],
            0.0)
        return jnp.dot(hh, w2[...], preferred_element_type=jnp.float32) + b2[...]

    r = jax.nn.sigmoid(mlp(gi, rw1_ref, rb1_ref, rw2_ref, rb2_ref))
    z = jax.nn.sigmoid(mlp(gi, zw1_ref, zb1_ref, zw2_ref, zb2_ref))
    ci = jnp.concatenate([r * x, mf, mr], axis=-1)
    cand = jnp.tanh(mlp(ci, cw1_ref, cb1_ref, cw2_ref, cb2_ref))
    fin_ref[...] = (1.0 - z) * x + z * cand
    z_ref[...] = z
    r_ref[...] = r


def _gru(x, h1f, hpf, h1r, hpr, rw1, rb1, rw2, rb2, zw1, zb1, zw2, zb2,
         cw1, cb1, cw2, cb2):
    full = lambda shape: pl.BlockSpec(shape, lambda i: (0,) * len(shape))
    nb = lambda w: pl.BlockSpec((_NBLK, w), lambda i: (i, 0))
    pb = pl.BlockSpec((2, _NBLK, H), lambda i: (0, i, 0))
    return pl.pallas_call(
        _gru_body,
        grid=(N // _NBLK,),
        in_specs=[nb(H), nb(H), pb, nb(H), pb,
                  full((GW, GW)), full((1, GW)), full((GW, H)), full((1, H)),
                  full((GW, GW)), full((1, GW)), full((GW, H)), full((1, H)),
                  full((GW, GW)), full((1, GW)), full((GW, H)), full((1, H))],
        out_specs=[nb(H), nb(H), nb(H)],
        out_shape=[jax.ShapeDtypeStruct((N, H), jnp.float32)] * 3,
    )(x, h1f, hpf, h1r, hpr,
      rw1.T, rb1.reshape(1, GW), rw2.T, rb2.reshape(1, H),
      zw1.T, zb1.reshape(1, GW), zw2.T, zb2.reshape(1, H),
      cw1.T, cb1.reshape(1, GW), cw2.T, cb2.reshape(1, H))


_LANES = None


CH2 = 64


def _gat_body(p1_h, p2_h, a_h, b_h, g1_h, g2_h,
              a_v, b_v, st1, st2, st1b, st2b, sem):
    cid = lax.axis_index("c")
    sid = lax.axis_index("s")
    wid = sid * NC + cid
    base = pl.multiple_of(wid * EPW, 8)
    pbase = pl.multiple_of(wid * EPAD, 128)

    pltpu.sync_copy(a_h.at[pl.ds(pbase, EPAD)], a_v)
    pltpu.sync_copy(b_h.at[pl.ds(pbase, EPAD)], b_v)

    def _pair(tp, carry):
        e0 = pl.multiple_of(tp * 2 * CH2, 8)
        e1 = pl.multiple_of((tp * 2 + 1) * CH2, 8)
        ga1 = pltpu.async_copy(p1_h.at[a_v.at[pl.ds(e0, CH2)]], st1, sem)
        ga2 = pltpu.async_copy(p2_h.at[b_v.at[pl.ds(e0, CH2)]], st2, sem)
        gb1 = pltpu.async_copy(p1_h.at[a_v.at[pl.ds(e1, CH2)]], st1b, sem)
        gb2 = pltpu.async_copy(p2_h.at[b_v.at[pl.ds(e1, CH2)]], st2b, sem)
        ga1.wait(); ga2.wait()
        pltpu.sync_copy(st1, g1_h.at[pl.ds(base + e0, CH2)])
        pltpu.sync_copy(st2, g2_h.at[pl.ds(base + e0, CH2)])
        gb1.wait(); gb2.wait()
        pltpu.sync_copy(st1b, g1_h.at[pl.ds(base + e1, CH2)])
        pltpu.sync_copy(st2b, g2_h.at[pl.ds(base + e1, CH2)])
        return carry
    lax.fori_loop(0, EPW // CH2 // 2, _pair, 0)

    ebase = (EPW // CH2) * CH2
    pltpu.async_copy(p1_h.at[a_v.at[pl.ds(ebase, L)]],
                     st1.at[pl.ds(0, L)], sem).wait()
    pltpu.async_copy(p2_h.at[b_v.at[pl.ds(ebase, L)]],
                     st2.at[pl.ds(0, L)], sem).wait()
    pltpu.sync_copy(st1.at[pl.ds(0, EPI)], g1_h.at[pl.ds(base + ebase, EPI)])
    pltpu.sync_copy(st2.at[pl.ds(0, EPI)], g2_h.at[pl.ds(base + ebase, EPI)])


def _gat(p1, p2, a_p, b_p):
    return pl.kernel(
        _gat_body,
        out_type=[jax.ShapeDtypeStruct((E, W), jnp.float32),
                  jax.ShapeDtypeStruct((E, W), jnp.float32)],
        mesh=_sc_mesh(),
        compiler_params=pltpu.CompilerParams(needs_layout_passes=False),
        scratch_types=[
            pltpu.VMEM((EPAD,), jnp.int32),
            pltpu.VMEM((EPAD,), jnp.int32),
            pltpu.VMEM((CH2, W), jnp.float32),
            pltpu.VMEM((CH2, W), jnp.float32),
            pltpu.VMEM((CH2, W), jnp.float32),
            pltpu.VMEM((CH2, W), jnp.float32),
            pltpu.SemaphoreType.DMA,
        ],
    )(p1, p2, a_p, b_p)


def _tscore_body(g1_ref, g2_ref, ef_ref, we_ref, b1_ref, w2_ref, b2_ref,
                 out_ref, *, is_fwd):
    pre = (g1_ref[...] + g2_ref[...]
           + jnp.dot(ef_ref[...], we_ref[...],
                     preferred_element_type=jnp.float32)
           + b1_ref[...])
    t = jnp.sum(jnp.maximum(pre, 0.0) * w2_ref[...], axis=-1, keepdims=True) \
        + b2_ref[0, 0]
    if is_fwd:
        t = jnp.where(t >= 0.0, t, 0.01 * t)
        out_ref[...] = jnp.exp(t * INV_TEMP)
    else:
        out_ref[...] = 1.0 / (1.0 + jnp.exp(-t))


def _tscore(g1, g2, ef, we, b1, w2, b2, is_fwd):
    full = lambda shape: pl.BlockSpec(shape, lambda i: (0, 0))
    eb = lambda w: pl.BlockSpec((_EBLK, w), lambda i: (i, 0))
    out = pl.pallas_call(
        functools.partial(_tscore_body, is_fwd=is_fwd),
        grid=(E // _EBLK,),
        in_specs=[eb(W), eb(W), eb(EF), full((EF, W)), full((1, W)),
                  full((1, W)), full((1, 1))],
        out_specs=pl.BlockSpec((_EBLK, 1), lambda i: (i, 0)),
        out_shape=jax.ShapeDtypeStruct((E, 1), jnp.float32),
    )(g1, g2, ef, we, b1.reshape(1, W), w2.reshape(1, W), b2.reshape(1, 1))
    return out[:, 0]


def _sden_body(e_h, dst_h, sp_h, e_v, dst_v, s_loc):
    cid = lax.axis_index("c")
    sid = lax.axis_index("s")
    wid = sid * NC + cid
    pbase = pl.multiple_of(wid * EPAD, 128)
    lanes = lax.iota(jnp.int32, L)

    pltpu.sync_copy(e_h.at[pl.ds(pbase, EPAD)], e_v)
    pltpu.sync_copy(dst_h.at[pl.ds(pbase, EPAD)], dst_v)

    zero16 = jnp.zeros((L,), jnp.float32)

    def _zero(k, carry):
        s_loc[pl.ds(pl.multiple_of(k * L, 8), L)] = zero16
        return carry
    lax.fori_loop(0, SPADN // L, _zero, 0)

    def _grp(g, carry):
        off = pl.multiple_of(g * L, 8)
        ev = e_v[pl.ds(off, L)]
        d16 = dst_v[pl.ds(off, L)]
        for k in range(L):
            dk = jnp.full((L,), d16[k], jnp.int32)
            plsc.addupdate_scatter(s_loc, [dk], ev, mask=(lanes == k))
        return carry
    lax.fori_loop(0, EPW // L, _grp, 0)

    ebase = (EPW // L) * L
    ev = e_v[pl.ds(ebase, L)]
    d16 = jnp.where(lanes < EPI, dst_v[pl.ds(ebase, L)], DUMMY)
    for k in range(EPI):
        dk = jnp.full((L,), d16[k], jnp.int32)
        plsc.addupdate_scatter(s_loc, [dk], ev, mask=(lanes == k))

    pltpu.sync_copy(s_loc, sp_h.at[pl.ds(pl.multiple_of(wid * SPADN, 128),
                                         SPADN)])


def _sden(e_p, dst_p):
    return pl.kernel(
        _sden_body,
        out_type=jax.ShapeDtypeStruct((NW * SPADN,), jnp.float32),
        mesh=_sc_mesh(),
        compiler_params=pltpu.CompilerParams(needs_layout_passes=False),
        scratch_types=[
            pltpu.VMEM((EPAD,), jnp.float32),
            pltpu.VMEM((EPAD,), jnp.int32),
            pltpu.VMEM((SPADN,), jnp.float32),
        ],
    )(e_p, dst_p)


_ZROWS = 16


def _scale_scatter(hrows, w16, row0, idx16, h_acc, sem):
    for k in range(L):
        wk = w16[k]
        for rr in range(H // L):
            sl = (row0 + k, pl.ds(rr * L, L))
            hrows[sl] = hrows[sl] * wk
    return pltpu.async_copy(hrows.at[pl.ds(row0, L)], h_acc.at[idx16], sem,
                            add=True)


def _zero_acc(zbuf, h_acc, sid):
    zero16 = jnp.zeros((L,), jnp.float32)

    def _zb(i, carry):
        for jj in range(H // L):
            zbuf[i, pl.ds(jj * L, L)] = zero16
        return carry
    lax.fori_loop(0, _ZROWS, _zb, 0)
    stripe = sid * (NPAD // NS)
    for q in range(NPAD // NS // _ZROWS):
        pltpu.sync_copy(zbuf, h_acc.at[pl.ds(stripe + q * _ZROWS, _ZROWS)])
    pltpu.sync_copy(zbuf, h_acc.at[pl.ds(stripe + NPAD // NS - _ZROWS,
                                         _ZROWS)])


def _f2_body(e_h, sp_h, hn_h, src_h, dst_h,
             w_h, hp_h,
             e_v, src_v, dst_v, s_tot, s_tmp, hrows, w_v, zbuf, h_acc,
             sem, sem2):
    cid = lax.axis_index("c")
    sid = lax.axis_index("s")
    wid = sid * NC + cid
    base = pl.multiple_of(wid * EPW, 8)
    pbase = pl.multiple_of(wid * EPAD, 128)
    lanes = lax.iota(jnp.int32, L)

    pltpu.sync_copy(e_h.at[pl.ds(pbase, EPAD)], e_v)
    pltpu.sync_copy(src_h.at[pl.ds(pbase, EPAD)], src_v)
    pltpu.sync_copy(dst_h.at[pl.ds(pbase, EPAD)], dst_v)

    _zero_acc(zbuf, h_acc, sid)
    plsc.subcore_barrier()

    pltpu.sync_copy(sp_h.at[pl.ds(0, SPADN)], s_tot)
    for rpart in range(1, NW):
        pltpu.sync_copy(sp_h.at[pl.ds(rpart * SPADN, SPADN)], s_tmp)

        def _acc(k2, carry):
            off = pl.ds(pl.multiple_of(k2 * L, 8), L)
            s_tot[off] = s_tot[off] + s_tmp[off]
            return carry
        lax.fori_loop(0, SPADN // L, _acc, 0)

    def _chunk(t, carry):
        eb = pl.multiple_of(t * CH, 8)
        pltpu.async_copy(hn_h.at[src_v.at[pl.ds(eb, CH)]], hrows, sem).wait()
        scat = []
        for g in range(CH // L):
            d16 = dst_v[pl.ds(eb + g * L, L)]
            e16 = e_v[pl.ds(eb + g * L, L)]
            s16 = plsc.load_gather(s_tot, [d16])
            w16 = e16 / (s16 + 1e-9)
            w_v[pl.ds(eb + g * L, L)] = w16
            scat.append(_scale_scatter(hrows, w16, g * L, d16, h_acc, sem2))
        for sc in scat:
            sc.wait()
        return carry
    lax.fori_loop(0, NCHUNK, _chunk, 0)

    ebase = NCHUNK * CH
    msk = lanes < EPI
    vsrc = jnp.where(msk, src_v[pl.ds(ebase, L)], 0)
    vdst_raw = dst_v[pl.ds(ebase, L)]
    vdst = jnp.where(msk, vdst_raw, 0)
    pltpu.async_copy(hn_h.at[vsrc], hrows.at[pl.ds(0, L)], sem).wait()
    e16 = e_v[pl.ds(ebase, L)]
    s16 = plsc.load_gather(s_tot, [vdst])
    w16 = e16 / (s16 + 1e-9)
    w_v[pl.ds(ebase, L)] = w16
    dsc = jnp.where(msk, vdst_raw, DUMMY)
    _scale_scatter(hrows, w16, 0, dsc, h_acc, sem2).wait()

    pltpu.sync_copy(w_v, w_h.at[pl.ds(pbase, EPAD)])
    plsc.subcore_barrier()

    @pl.when(sid == 0)
    def _flush():
        pltpu.sync_copy(h_acc.at[pl.ds(0, N)], hp_h.at[cid])


def _f2(e, sp, hn, srci, dsti):
    return pl.kernel(
        _f2_body,
        out_type=[jax.ShapeDtypeStruct((EFULL,), jnp.float32),
                  jax.ShapeDtypeStruct((2, N, H), jnp.float32)],
        mesh=_sc_mesh(),
        compiler_params=pltpu.CompilerParams(needs_layout_passes=False),
        scratch_types=[
            pltpu.VMEM((EPAD,), jnp.float32),
            pltpu.VMEM((EPAD,), jnp.int32),
            pltpu.VMEM((EPAD,), jnp.int32),
            pltpu.VMEM((SPADN,), jnp.float32),
            pltpu.VMEM((SPADN,), jnp.float32),
            pltpu.VMEM((CH, H), jnp.float32),
            pltpu.VMEM((EPAD,), jnp.float32),
            pltpu.VMEM((_ZROWS, H), jnp.float32),
            pltpu.VMEM_SHARED((NPAD, H), jnp.float32),
            pltpu.SemaphoreType.DMA,
            pltpu.SemaphoreType.DMA,
        ],
    )(e, sp, hn, srci, dsti)


def _rsc_body(g_h, hn_h, gat_h, sct_h, hp_h,
              g_v, gat_v, sct_v, hrows, zbuf, h_acc, sem, sem2):
    cid = lax.axis_index("c")
    sid = lax.axis_index("s")
    wid = sid * NC + cid
    pbase = pl.multiple_of(wid * EPAD, 128)
    lanes = lax.iota(jnp.int32, L)

    pltpu.sync_copy(g_h.at[pl.ds(pbase, EPAD)], g_v)
    pltpu.sync_copy(gat_h.at[pl.ds(pbase, EPAD)], gat_v)
    pltpu.sync_copy(sct_h.at[pl.ds(pbase, EPAD)], sct_v)

    _zero_acc(zbuf, h_acc, sid)
    plsc.subcore_barrier()

    def _chunk(t, carry):
        eb = pl.multiple_of(t * CH, 8)
        pltpu.async_copy(hn_h.at[gat_v.at[pl.ds(eb, CH)]], hrows, sem).wait()
        scat = []
        for g in range(CH // L):
            w16 = g_v[pl.ds(eb + g * L, L)]
            i16 = sct_v[pl.ds(eb + g * L, L)]
            scat.append(_scale_scatter(hrows, w16, g * L, i16, h_acc, sem2))
        for sc in scat:
            sc.wait()
        return carry
    lax.fori_loop(0, NCHUNK, _chunk, 0)

    ebase = NCHUNK * CH
    msk = lanes < EPI
    vgat = jnp.where(msk, gat_v[pl.ds(ebase, L)], 0)
    pltpu.async_copy(hn_h.at[vgat], hrows.at[pl.ds(0, L)], sem).wait()
    w16 = g_v[pl.ds(ebase, L)]
    ssc = jnp.where(msk, sct_v[pl.ds(ebase, L)], DUMMY)
    _scale_scatter(hrows, w16, 0, ssc, h_acc, sem2).wait()

    plsc.subcore_barrier()

    @pl.when(sid == 0)
    def _flush():
        pltpu.sync_copy(h_acc.at[pl.ds(0, N)], hp_h.at[cid])


def _rsc(g_p, hn, gat_p, sct_p):
    return pl.kernel(
        _rsc_body,
        out_type=jax.ShapeDtypeStruct((2, N, H), jnp.float32),
        mesh=_sc_mesh(),
        compiler_params=pltpu.CompilerParams(needs_layout_passes=False),
        scratch_types=[
            pltpu.VMEM((EPAD,), jnp.float32),
            pltpu.VMEM((EPAD,), jnp.int32),
            pltpu.VMEM((EPAD,), jnp.int32),
            pltpu.VMEM((CH, H), jnp.float32),
            pltpu.VMEM((_ZROWS, H), jnp.float32),
            pltpu.VMEM_SHARED((NPAD, H), jnp.float32),
            pltpu.SemaphoreType.DMA,
            pltpu.SemaphoreType.DMA,
        ],
    )(g_p, hn, gat_p, sct_p)


def kernel(x, x_s, edge_index, edge_features, fwd_W1, fwd_b1, fwd_W2, fwd_b2,
           rev_W1, rev_b1, rev_W2, rev_b2, ln_w, ln_b, r_W1, r_b1, r_W2, r_b2,
           z_W1, z_b1, z_W2, z_b2, c_W1, c_b1, c_W2, c_b2):
    src = edge_index[0].astype(jnp.int32)
    dst = edge_index[1].astype(jnp.int32)
    pad = lambda a: jnp.pad(a.reshape(NW, EPW),
                            ((0, 0), (0, EPAD - EPW))).reshape(-1)
    unpad = lambda a: a.reshape(NW, EPAD)[:, :EPW].reshape(E)
    src_p, dst_p = pad(src), pad(dst)


    def wslices(w1):
        return (w1[:, :H].T, w1[:, W:W + S].T,
                w1[:, H:W].T, w1[:, W + S:W + 2 * S].T)

    hn0, p1, p2 = _proj_first(x, x_s, ln_w, ln_b, *wslices(fwd_W1[0]))
    hn0r, q1, q2 = _proj_first(x, x_s, ln_w, ln_b, *wslices(rev_W1[0]))
    g1a, g2a = _gat(p1, p2, src_p, dst_p)
    r1a, r2a = _gat(q1, q2, dst_p, src_p)
    e0 = pad(_tscore(g1a, g2a, edge_features, fwd_W1[0, :, W + 2 * S:].T,
                 fwd_b1[0], fwd_W2[0, 0], fwd_b2[0], True))
    ge0 = pad(_tscore(r1a, r2a, edge_features, rev_W1[0, :, W + 2 * S:].T,
                  rev_b1[0], rev_W2[0, 0], rev_b2[0], False))
    sp0 = _sden(e0, dst_p)
    w0, hp0 = _f2(e0, sp0, hn0, src_p, dst_p)
    hq0 = _rsc(ge0, hn0r, dst_p, src_p)

    hn1, p1b, p2b, h1f = _proj_next(x, hp0, x_s, ln_w, ln_b,
                                    *wslices(fwd_W1[1]))
    hn1r, q1c, q2c, h1r = _proj_next(x, hq0, x_s, ln_w, ln_b,
                                     *wslices(rev_W1[1]))
    g1b, g2b = _gat(p1b, p2b, src_p, dst_p)
    r1b, r2b = _gat(q1c, q2c, dst_p, src_p)
    e1 = pad(_tscore(g1b, g2b, edge_features, fwd_W1[1, :, W + 2 * S:].T,
                 fwd_b1[1], fwd_W2[1, 0], fwd_b2[1], True))
    ge1 = pad(_tscore(r1b, r2b, edge_features, rev_W1[1, :, W + 2 * S:].T,
                  rev_b1[1], rev_W2[1, 0], rev_b2[1], False))
    sp1 = _sden(e1, dst_p)
    w1, hp1 = _f2(e1, sp1, hn1, src_p, dst_p)
    hq1 = _rsc(ge1, hn1r, dst_p, src_p)
    fwd_ws = jnp.stack([unpad(w0), unpad(w1)], axis=-1)
    rev_ws = jnp.stack([unpad(ge0), unpad(ge1)], axis=-1)

    final, z, r = _gru(x, h1f, hp1, h1r, hq1,
                       r_W1, r_b1, r_W2, r_b2,
                       z_W1, z_b1, z_W2, z_b2,
                       c_W1, c_b1, c_W2, c_b2)
    return (final, fwd_ws, rev_ws, z, r)


# speedup vs baseline: 1.3328x; 1.1944x over previous
import functools
import math

import jax
import jax.numpy as jnp
from jax import lax
from jax.experimental import pallas as pl
from jax.experimental.pallas import tpu as pltpu
from jax.experimental.pallas import tpu_sc as plsc

N = 10000
E = 160000
H = 128
S = 16
EF = 16
K = 2
W = 2 * H
GW = 3 * H

NC = 2
NS = 16
NW = NC * NS
L = 16
EPW = E // NW
CH = 32
NCHUNK = (EPW // CH)
EPI = EPW - NCHUNK * CH
DUMMY = N
NPAD = N + L
EPAD = 5120
EFULL = NW * EPAD
SPADN = 10112
INV_TEMP = 1.0 / math.sqrt(float(H))

_NBLK = 1000
_EBLK = 2000

@functools.cache
def _sc_mesh():
    return plsc.VectorSubcoreMesh(
        core_axis_name="c", subcore_axis_name="s",
        num_cores=NC, num_subcores=NS)


def _ln(h, lnw, lnb):
    mu = jnp.mean(h, axis=-1, keepdims=True)
    var = jnp.mean((h - mu) * (h - mu), axis=-1, keepdims=True)
    return (h - mu) * lax.rsqrt(var + 1e-5) * lnw + lnb


def _proj_first_body(x_ref, xs_ref, lnw_ref, lnb_ref, wh1_ref, ws1_ref,
                     wh2_ref, ws2_ref, hn_ref, p1_ref, p2_ref):
    hn = _ln(x_ref[...], lnw_ref[...], lnb_ref[...])
    hn_ref[...] = hn
    xs = xs_ref[...]
    p1_ref[...] = (jnp.dot(hn, wh1_ref[...], preferred_element_type=jnp.float32)
                   + jnp.dot(xs, ws1_ref[...], preferred_element_type=jnp.float32))
    p2_ref[...] = (jnp.dot(hn, wh2_ref[...], preferred_element_type=jnp.float32)
                   + jnp.dot(xs, ws2_ref[...], preferred_element_type=jnp.float32))


def _proj_next_body(x_ref, hp_ref, xs_ref, lnw_ref, lnb_ref, wh1_ref, ws1_ref,
                    wh2_ref, ws2_ref, hn_ref, p1_ref, p2_ref, h_ref):
    h = x_ref[...] + hp_ref[0] + hp_ref[1]
    h_ref[...] = h
    hn = _ln(h, lnw_ref[...], lnb_ref[...])
    hn_ref[...] = hn
    xs = xs_ref[...]
    p1_ref[...] = (jnp.dot(hn, wh1_ref[...], preferred_element_type=jnp.float32)
                   + jnp.dot(xs, ws1_ref[...], preferred_element_type=jnp.float32))
    p2_ref[...] = (jnp.dot(hn, wh2_ref[...], preferred_element_type=jnp.float32)
                   + jnp.dot(xs, ws2_ref[...], preferred_element_type=jnp.float32))


def _proj_specs():
    full = lambda shape: pl.BlockSpec(shape, lambda i: (0,) * len(shape))
    nb = lambda w: pl.BlockSpec((_NBLK, w), lambda i: (i, 0))
    return full, nb


def _proj_first(x, xs, lnw, lnb, wh1, ws1, wh2, ws2):
    full, nb = _proj_specs()
    return pl.pallas_call(
        _proj_first_body,
        grid=(N // _NBLK,),
        in_specs=[nb(H), nb(S), full((1, H)), full((1, H)),
                  full((H, W)), full((S, W)), full((H, W)), full((S, W))],
        out_specs=[nb(H), nb(W), nb(W)],
        out_shape=[jax.ShapeDtypeStruct((N, H), jnp.float32),
                   jax.ShapeDtypeStruct((N, W), jnp.float32),
                   jax.ShapeDtypeStruct((N, W), jnp.float32)],
    )(x, xs, lnw.reshape(1, H), lnb.reshape(1, H), wh1, ws1, wh2, ws2)


def _proj_next(x, hp, xs, lnw, lnb, wh1, ws1, wh2, ws2):
    full, nb = _proj_specs()
    return pl.pallas_call(
        _proj_next_body,
        grid=(N // _NBLK,),
        in_specs=[nb(H), pl.BlockSpec((2, _NBLK, H), lambda i: (0, i, 0)),
                  nb(S), full((1, H)), full((1, H)),
                  full((H, W)), full((S, W)), full((H, W)), full((S, W))],
        out_specs=[nb(H), nb(W), nb(W), nb(H)],
        out_shape=[jax.ShapeDtypeStruct((N, H), jnp.float32),
                   jax.ShapeDtypeStruct((N, W), jnp.float32),
                   jax.ShapeDtypeStruct((N, W), jnp.float32),
                   jax.ShapeDtypeStruct((N, H), jnp.float32)],
    )(x, hp, xs, lnw.reshape(1, H), lnb.reshape(1, H), wh1, ws1, wh2, ws2)


def _gru_body(x_ref, h1f_ref, hpf_ref, h1r_ref, hpr_ref,
              rw1_ref, rb1_ref, rw2_ref, rb2_ref,
              zw1_ref, zb1_ref, zw2_ref, zb2_ref,
              cw1_ref, cb1_ref, cw2_ref, cb2_ref,
              fin_ref, z_ref, r_ref):
    x = x_ref[...]
    mf = h1f_ref[...] + hpf_ref[0] + hpf_ref[1] - x
    mr = h1r_ref[...] + hpr_ref[0] + hpr_ref[1] - x
    gi = jnp.concatenate([x, mf, mr], axis=-1)

    def mlp(inp, w1, b1, w2, b2):
        hh = jnp.maximum(
            jnp.dot(inp, w1[...], preferred_element_type=jnp.float32) + b1[...],
            0.0)
        return jnp.dot(hh, w2[...], preferred_element_type=jnp.float32) + b2[...]

    r = jax.nn.sigmoid(mlp(gi, rw1_ref, rb1_ref, rw2_ref, rb2_ref))
    z = jax.nn.sigmoid(mlp(gi, zw1_ref, zb1_ref, zw2_ref, zb2_ref))
    ci = jnp.concatenate([r * x, mf, mr], axis=-1)
    cand = jnp.tanh(mlp(ci, cw1_ref, cb1_ref, cw2_ref, cb2_ref))
    fin_ref[...] = (1.0 - z) * x + z * cand
    z_ref[...] = z
    r_ref[...] = r


def _gru(x, h1f, hpf, h1r, hpr, rw1, rb1, rw2, rb2, zw1, zb1, zw2, zb2,
         cw1, cb1, cw2, cb2):
    full = lambda shape: pl.BlockSpec(shape, lambda i: (0,) * len(shape))
    nb = lambda w: pl.BlockSpec((_NBLK, w), lambda i: (i, 0))
    pb = pl.BlockSpec((2, _NBLK, H), lambda i: (0, i, 0))
    return pl.pallas_call(
        _gru_body,
        grid=(N // _NBLK,),
        in_specs=[nb(H), nb(H), pb, nb(H), pb,
                  full((GW, GW)), full((1, GW)), full((GW, H)), full((1, H)),
                  full((GW, GW)), full((1, GW)), full((GW, H)), full((1, H)),
                  full((GW, GW)), full((1, GW)), full((GW, H)), full((1, H))],
        out_specs=[nb(H), nb(H), nb(H)],
        out_shape=[jax.ShapeDtypeStruct((N, H), jnp.float32)] * 3,
    )(x, h1f, hpf, h1r, hpr,
      rw1.T, rb1.reshape(1, GW), rw2.T, rb2.reshape(1, H),
      zw1.T, zb1.reshape(1, GW), zw2.T, zb2.reshape(1, H),
      cw1.T, cb1.reshape(1, GW), cw2.T, cb2.reshape(1, H))


_LANES = None


CH2 = 64


def _gat_body(p1_h, p2_h, a_h, b_h, g1_h, g2_h,
              a_v, b_v, st1, st2, st1b, st2b, sem):
    cid = lax.axis_index("c")
    sid = lax.axis_index("s")
    wid = sid * NC + cid
    base = pl.multiple_of(wid * EPW, 8)
    pbase = pl.multiple_of(wid * EPAD, 128)

    pltpu.sync_copy(a_h.at[pl.ds(pbase, EPAD)], a_v)
    pltpu.sync_copy(b_h.at[pl.ds(pbase, EPAD)], b_v)

    def _pair(tp, carry):
        e0 = pl.multiple_of(tp * 2 * CH2, 8)
        e1 = pl.multiple_of((tp * 2 + 1) * CH2, 8)
        ga1 = pltpu.async_copy(p1_h.at[a_v.at[pl.ds(e0, CH2)]], st1, sem)
        ga2 = pltpu.async_copy(p2_h.at[b_v.at[pl.ds(e0, CH2)]], st2, sem)
        gb1 = pltpu.async_copy(p1_h.at[a_v.at[pl.ds(e1, CH2)]], st1b, sem)
        gb2 = pltpu.async_copy(p2_h.at[b_v.at[pl.ds(e1, CH2)]], st2b, sem)
        ga1.wait(); ga2.wait()
        pltpu.sync_copy(st1, g1_h.at[pl.ds(base + e0, CH2)])
        pltpu.sync_copy(st2, g2_h.at[pl.ds(base + e0, CH2)])
        gb1.wait(); gb2.wait()
        pltpu.sync_copy(st1b, g1_h.at[pl.ds(base + e1, CH2)])
        pltpu.sync_copy(st2b, g2_h.at[pl.ds(base + e1, CH2)])
        return carry
    lax.fori_loop(0, EPW // CH2 // 2, _pair, 0)

    ebase = (EPW // CH2) * CH2
    pltpu.async_copy(p1_h.at[a_v.at[pl.ds(ebase, L)]],
                     st1.at[pl.ds(0, L)], sem).wait()
    pltpu.async_copy(p2_h.at[b_v.at[pl.ds(ebase, L)]],
                     st2.at[pl.ds(0, L)], sem).wait()
    pltpu.sync_copy(st1.at[pl.ds(0, EPI)], g1_h.at[pl.ds(base + ebase, EPI)])
    pltpu.sync_copy(st2.at[pl.ds(0, EPI)], g2_h.at[pl.ds(base + ebase, EPI)])


def _gat(p1, p2, a_p, b_p):
    return pl.kernel(
        _gat_body,
        out_type=[jax.ShapeDtypeStruct((E, W), jnp.float32),
                  jax.ShapeDtypeStruct((E, W), jnp.float32)],
        mesh=_sc_mesh(),
        compiler_params=pltpu.CompilerParams(needs_layout_passes=False),
        scratch_types=[
            pltpu.VMEM((EPAD,), jnp.int32),
            pltpu.VMEM((EPAD,), jnp.int32),
            pltpu.VMEM((CH2, W), jnp.float32),
            pltpu.VMEM((CH2, W), jnp.float32),
            pltpu.VMEM((CH2, W), jnp.float32),
            pltpu.VMEM((CH2, W), jnp.float32),
            pltpu.SemaphoreType.DMA,
        ],
    )(p1, p2, a_p, b_p)


def _tscore_body(g1_ref, g2_ref, ef_ref, we_ref, b1_ref, w2_ref, b2_ref,
                 out_ref, *, is_fwd):
    pre = (g1_ref[...] + g2_ref[...]
           + jnp.dot(ef_ref[...], we_ref[...],
                     preferred_element_type=jnp.float32)
           + b1_ref[...])
    t = jnp.sum(jnp.maximum(pre, 0.0) * w2_ref[...], axis=-1, keepdims=True) \
        + b2_ref[0, 0]
    if is_fwd:
        t = jnp.where(t >= 0.0, t, 0.01 * t)
        out_ref[...] = jnp.exp(t * INV_TEMP)
    else:
        out_ref[...] = 1.0 / (1.0 + jnp.exp(-t))


def _tscore(g1, g2, ef, we, b1, w2, b2, is_fwd):
    full = lambda shape: pl.BlockSpec(shape, lambda i: (0, 0))
    eb = lambda w: pl.BlockSpec((_EBLK, w), lambda i: (i, 0))
    out = pl.pallas_call(
        functools.partial(_tscore_body, is_fwd=is_fwd),
        grid=(E // _EBLK,),
        in_specs=[eb(W), eb(W), eb(EF), full((EF, W)), full((1, W)),
                  full((1, W)), full((1, 1))],
        out_specs=pl.BlockSpec((_EBLK, 1), lambda i: (i, 0)),
        out_shape=jax.ShapeDtypeStruct((E, 1), jnp.float32),
    )(g1, g2, ef, we, b1.reshape(1, W), w2.reshape(1, W), b2.reshape(1, 1))
    return out[:, 0]


def _sred_body(sp_ref, out_ref):
    out_ref[...] = jnp.sum(sp_ref[...], axis=0, keepdims=True)


def _sred(sp):
    out = pl.pallas_call(
        _sred_body,
        grid=(1,),
        in_specs=[pl.BlockSpec((NW, SPADN), lambda i: (0, 0))],
        out_specs=pl.BlockSpec((1, SPADN), lambda i: (0, 0)),
        out_shape=jax.ShapeDtypeStruct((1, SPADN), jnp.float32),
    )(sp.reshape(NW, SPADN))
    return out[0]


def _sden_body(e_h, dst_h, sp_h, e_v, dst_v, s_loc):
    cid = lax.axis_index("c")
    sid = lax.axis_index("s")
    wid = sid * NC + cid
    pbase = pl.multiple_of(wid * EPAD, 128)
    lanes = lax.iota(jnp.int32, L)

    pltpu.sync_copy(e_h.at[pl.ds(pbase, EPAD)], e_v)
    pltpu.sync_copy(dst_h.at[pl.ds(pbase, EPAD)], dst_v)

    zero16 = jnp.zeros((L,), jnp.float32)

    def _zero(k, carry):
        s_loc[pl.ds(pl.multiple_of(k * L, 8), L)] = zero16
        return carry
    lax.fori_loop(0, SPADN // L, _zero, 0)

    def _grp(g, carry):
        off = pl.multiple_of(g * L, 8)
        ev = e_v[pl.ds(off, L)]
        d16 = dst_v[pl.ds(off, L)]
        for k in range(L):
            dk = jnp.full((L,), d16[k], jnp.int32)
            plsc.addupdate_scatter(s_loc, [dk], ev, mask=(lanes == k))
        return carry
    lax.fori_loop(0, EPW // L, _grp, 0)

    ebase = (EPW // L) * L
    ev = e_v[pl.ds(ebase, L)]
    d16 = jnp.where(lanes < EPI, dst_v[pl.ds(ebase, L)], DUMMY)
    for k in range(EPI):
        dk = jnp.full((L,), d16[k], jnp.int32)
        plsc.addupdate_scatter(s_loc, [dk], ev, mask=(lanes == k))

    pltpu.sync_copy(s_loc, sp_h.at[pl.ds(pl.multiple_of(wid * SPADN, 128),
                                         SPADN)])


def _sden(e_p, dst_p):
    return pl.kernel(
        _sden_body,
        out_type=jax.ShapeDtypeStruct((NW * SPADN,), jnp.float32),
        mesh=_sc_mesh(),
        compiler_params=pltpu.CompilerParams(needs_layout_passes=False),
        scratch_types=[
            pltpu.VMEM((EPAD,), jnp.float32),
            pltpu.VMEM((EPAD,), jnp.int32),
            pltpu.VMEM((SPADN,), jnp.float32),
        ],
    )(e_p, dst_p)


_ZROWS = 16


def _scale_scatter(hrows, w16, row0, idx16, h_acc, sem):
    for k in range(L):
        wk = w16[k]
        for rr in range(H // L):
            sl = (row0 + k, pl.ds(rr * L, L))
            hrows[sl] = hrows[sl] * wk
    return pltpu.async_copy(hrows.at[pl.ds(row0, L)], h_acc.at[idx16], sem,
                            add=True)


def _zero_acc(zbuf, h_acc, sid):
    zero16 = jnp.zeros((L,), jnp.float32)

    def _zb(i, carry):
        for jj in range(H // L):
            zbuf[i, pl.ds(jj * L, L)] = zero16
        return carry
    lax.fori_loop(0, _ZROWS, _zb, 0)
    stripe = sid * (NPAD // NS)
    for q in range(NPAD // NS // _ZROWS):
        pltpu.sync_copy(zbuf, h_acc.at[pl.ds(stripe + q * _ZROWS, _ZROWS)])
    pltpu.sync_copy(zbuf, h_acc.at[pl.ds(stripe + NPAD // NS - _ZROWS,
                                         _ZROWS)])


def _f2_body(e_h, sp_h, hn_h, src_h, dst_h,
             w_h, hp_h,
             e_v, src_v, dst_v, s_tot, hrows, hrows2, w_v, zbuf, h_acc,
             sem, sem2):
    cid = lax.axis_index("c")
    sid = lax.axis_index("s")
    wid = sid * NC + cid
    pbase = pl.multiple_of(wid * EPAD, 128)
    lanes = lax.iota(jnp.int32, L)

    pltpu.sync_copy(e_h.at[pl.ds(pbase, EPAD)], e_v)
    pltpu.sync_copy(src_h.at[pl.ds(pbase, EPAD)], src_v)
    pltpu.sync_copy(dst_h.at[pl.ds(pbase, EPAD)], dst_v)
    pltpu.sync_copy(sp_h, s_tot)

    _zero_acc(zbuf, h_acc, sid)
    plsc.subcore_barrier()

    def _half(eb, rows):
        scat = []
        for g in range(CH // L):
            d16 = dst_v[pl.ds(eb + g * L, L)]
            e16 = e_v[pl.ds(eb + g * L, L)]
            s16 = plsc.load_gather(s_tot, [d16])
            w16 = e16 / (s16 + 1e-9)
            w_v[pl.ds(eb + g * L, L)] = w16
            scat.append(_scale_scatter(rows, w16, g * L, d16, h_acc, sem2))
        return scat

    def _pair(tp, carry):
        e0 = pl.multiple_of(tp * 2 * CH, 8)
        e1 = pl.multiple_of((tp * 2 + 1) * CH, 8)
        ga = pltpu.async_copy(hn_h.at[src_v.at[pl.ds(e0, CH)]], hrows, sem)
        gb = pltpu.async_copy(hn_h.at[src_v.at[pl.ds(e1, CH)]], hrows2, sem)
        ga.wait()
        sa = _half(e0, hrows)
        gb.wait()
        sb = _half(e1, hrows2)
        for sc in sa + sb:
            sc.wait()
        return carry
    lax.fori_loop(0, NCHUNK // 2, _pair, 0)

    ebase = NCHUNK * CH
    msk = lanes < EPI
    vsrc = jnp.where(msk, src_v[pl.ds(ebase, L)], 0)
    vdst_raw = dst_v[pl.ds(ebase, L)]
    vdst = jnp.where(msk, vdst_raw, 0)
    pltpu.async_copy(hn_h.at[vsrc], hrows.at[pl.ds(0, L)], sem).wait()
    e16 = e_v[pl.ds(ebase, L)]
    s16 = plsc.load_gather(s_tot, [vdst])
    w16 = e16 / (s16 + 1e-9)
    w_v[pl.ds(ebase, L)] = w16
    dsc = jnp.where(msk, vdst_raw, DUMMY)
    _scale_scatter(hrows, w16, 0, dsc, h_acc, sem2).wait()

    pltpu.sync_copy(w_v, w_h.at[pl.ds(pbase, EPAD)])
    plsc.subcore_barrier()

    @pl.when(sid == 0)
    def _flush():
        pltpu.sync_copy(h_acc.at[pl.ds(0, N)], hp_h.at[cid])


def _f2(e, s_tot, hn, srci, dsti):
    return pl.kernel(
        _f2_body,
        out_type=[jax.ShapeDtypeStruct((EFULL,), jnp.float32),
                  jax.ShapeDtypeStruct((2, N, H), jnp.float32)],
        mesh=_sc_mesh(),
        compiler_params=pltpu.CompilerParams(needs_layout_passes=False),
        scratch_types=[
            pltpu.VMEM((EPAD,), jnp.float32),
            pltpu.VMEM((EPAD,), jnp.int32),
            pltpu.VMEM((EPAD,), jnp.int32),
            pltpu.VMEM((SPADN,), jnp.float32),
            pltpu.VMEM((CH, H), jnp.float32),
            pltpu.VMEM((CH, H), jnp.float32),
            pltpu.VMEM((EPAD,), jnp.float32),
            pltpu.VMEM((_ZROWS, H), jnp.float32),
            pltpu.VMEM_SHARED((NPAD, H), jnp.float32),
            pltpu.SemaphoreType.DMA,
            pltpu.SemaphoreType.DMA,
        ],
    )(e, s_tot, hn, srci, dsti)


def _rsc_body(g_h, hn_h, gat_h, sct_h, hp_h,
              g_v, gat_v, sct_v, hrows, hrows2, zbuf, h_acc, sem, sem2):
    cid = lax.axis_index("c")
    sid = lax.axis_index("s")
    wid = sid * NC + cid
    pbase = pl.multiple_of(wid * EPAD, 128)
    lanes = lax.iota(jnp.int32, L)

    pltpu.sync_copy(g_h.at[pl.ds(pbase, EPAD)], g_v)
    pltpu.sync_copy(gat_h.at[pl.ds(pbase, EPAD)], gat_v)
    pltpu.sync_copy(sct_h.at[pl.ds(pbase, EPAD)], sct_v)

    _zero_acc(zbuf, h_acc, sid)
    plsc.subcore_barrier()

    def _half(eb, rows):
        scat = []
        for g in range(CH // L):
            w16 = g_v[pl.ds(eb + g * L, L)]
            i16 = sct_v[pl.ds(eb + g * L, L)]
            scat.append(_scale_scatter(rows, w16, g * L, i16, h_acc, sem2))
        return scat

    def _pair(tp, carry):
        e0 = pl.multiple_of(tp * 2 * CH, 8)
        e1 = pl.multiple_of((tp * 2 + 1) * CH, 8)
        ga = pltpu.async_copy(hn_h.at[gat_v.at[pl.ds(e0, CH)]], hrows, sem)
        gb = pltpu.async_copy(hn_h.at[gat_v.at[pl.ds(e1, CH)]], hrows2, sem)
        ga.wait()
        sa = _half(e0, hrows)
        gb.wait()
        sb = _half(e1, hrows2)
        for sc in sa + sb:
            sc.wait()
        return carry
    lax.fori_loop(0, NCHUNK // 2, _pair, 0)

    ebase = NCHUNK * CH
    msk = lanes < EPI
    vgat = jnp.where(msk, gat_v[pl.ds(ebase, L)], 0)
    pltpu.async_copy(hn_h.at[vgat], hrows.at[pl.ds(0, L)], sem).wait()
    w16 = g_v[pl.ds(ebase, L)]
    ssc = jnp.where(msk, sct_v[pl.ds(ebase, L)], DUMMY)
    _scale_scatter(hrows, w16, 0, ssc, h_acc, sem2).wait()

    plsc.subcore_barrier()

    @pl.when(sid == 0)
    def _flush():
        pltpu.sync_copy(h_acc.at[pl.ds(0, N)], hp_h.at[cid])


def _rsc(g_p, hn, gat_p, sct_p):
    return pl.kernel(
        _rsc_body,
        out_type=jax.ShapeDtypeStruct((2, N, H), jnp.float32),
        mesh=_sc_mesh(),
        compiler_params=pltpu.CompilerParams(needs_layout_passes=False),
        scratch_types=[
            pltpu.VMEM((EPAD,), jnp.float32),
            pltpu.VMEM((EPAD,), jnp.int32),
            pltpu.VMEM((EPAD,), jnp.int32),
            pltpu.VMEM((CH, H), jnp.float32),
            pltpu.VMEM((CH, H), jnp.float32),
            pltpu.VMEM((_ZROWS, H), jnp.float32),
            pltpu.VMEM_SHARED((NPAD, H), jnp.float32),
            pltpu.SemaphoreType.DMA,
            pltpu.SemaphoreType.DMA,
        ],
    )(g_p, hn, gat_p, sct_p)


def kernel(x, x_s, edge_index, edge_features, fwd_W1, fwd_b1, fwd_W2, fwd_b2,
           rev_W1, rev_b1, rev_W2, rev_b2, ln_w, ln_b, r_W1, r_b1, r_W2, r_b2,
           z_W1, z_b1, z_W2, z_b2, c_W1, c_b1, c_W2, c_b2):
    src = edge_index[0].astype(jnp.int32)
    dst = edge_index[1].astype(jnp.int32)
    pad = lambda a: jnp.pad(a.reshape(NW, EPW),
                            ((0, 0), (0, EPAD - EPW))).reshape(-1)
    unpad = lambda a: a.reshape(NW, EPAD)[:, :EPW].reshape(E)
    src_p, dst_p = pad(src), pad(dst)


    def wslices(w1):
        return (w1[:, :H].T, w1[:, W:W + S].T,
                w1[:, H:W].T, w1[:, W + S:W + 2 * S].T)

    hn0, p1, p2 = _proj_first(x, x_s, ln_w, ln_b, *wslices(fwd_W1[0]))
    hn0r, q1, q2 = _proj_first(x, x_s, ln_w, ln_b, *wslices(rev_W1[0]))
    g1a, g2a = _gat(p1, p2, src_p, dst_p)
    r1a, r2a = _gat(q1, q2, dst_p, src_p)
    e0 = pad(_tscore(g1a, g2a, edge_features, fwd_W1[0, :, W + 2 * S:].T,
                 fwd_b1[0], fwd_W2[0, 0], fwd_b2[0], True))
    ge0 = pad(_tscore(r1a, r2a, edge_features, rev_W1[0, :, W + 2 * S:].T,
                  rev_b1[0], rev_W2[0, 0], rev_b2[0], False))
    sp0 = _sred(_sden(e0, dst_p))
    w0, hp0 = _f2(e0, sp0, hn0, src_p, dst_p)
    hq0 = _rsc(ge0, hn0r, dst_p, src_p)

    hn1, p1b, p2b, h1f = _proj_next(x, hp0, x_s, ln_w, ln_b,
                                    *wslices(fwd_W1[1]))
    hn1r, q1c, q2c, h1r = _proj_next(x, hq0, x_s, ln_w, ln_b,
                                     *wslices(rev_W1[1]))
    g1b, g2b = _gat(p1b, p2b, src_p, dst_p)
    r1b, r2b = _gat(q1c, q2c, dst_p, src_p)
    e1 = pad(_tscore(g1b, g2b, edge_features, fwd_W1[1, :, W + 2 * S:].T,
                 fwd_b1[1], fwd_W2[1, 0], fwd_b2[1], True))
    ge1 = pad(_tscore(r1b, r2b, edge_features, rev_W1[1, :, W + 2 * S:].T,
                  rev_b1[1], rev_W2[1, 0], rev_b2[1], False))
    sp1 = _sred(_sden(e1, dst_p))
    w1, hp1 = _f2(e1, sp1, hn1, src_p, dst_p)
    hq1 = _rsc(ge1, hn1r, dst_p, src_p)
    fwd_ws = jnp.stack([unpad(w0), unpad(w1)], axis=-1)
    rev_ws = jnp.stack([unpad(ge0), unpad(ge1)], axis=-1)

    final, z, r = _gru(x, h1f, hp1, h1r, hq1,
                       r_W1, r_b1, r_W2, r_b2,
                       z_W1, z_b1, z_W2, z_b2,
                       c_W1, c_b1, c_W2, c_b2)
    return (final, fwd_ws, rev_ws, z, r)
